# baseline (speedup 1.0000x reference)
import functools

import jax
import jax.numpy as jnp
from jax import lax
from jax.experimental import pallas as pl
from jax.experimental.pallas import tpu as pltpu
from jax.experimental.pallas import tpu_sc as plsc

_NC = 2
_NS = 16
_NW = _NC * _NS

_ROWS = 4096
_NB = 8
_F = 128
_CH = 8
_SLOTS = 4
_PER_W = _ROWS // _NW
_NCHUNK = _PER_W // _CH


def _sc_body(x1_hbm, x2_hbm, out_hbm, a_v, b_v, lsem, wsem):
    wid = lax.axis_index("s") * _NC + lax.axis_index("c")
    base = wid * _PER_W

    def start_loads(c):
        s = c % _SLOTS
        bk = base + c * _CH
        return [
            pltpu.async_copy(x1_hbm.at[pl.ds(bk, _CH)], a_v.at[s], lsem),
            pltpu.async_copy(x2_hbm.at[pl.ds(bk, _CH)], b_v.at[s], lsem),
        ]

    loads = {0: start_loads(0)}
    writes = {}
    for c in range(_NCHUNK):
        s = c % _SLOTS
        if c + 1 - _SLOTS >= 0:
            for d in writes[c + 1 - _SLOTS]:
                d.wait()
        if c + 1 < _NCHUNK:
            loads[c + 1] = start_loads(c + 1)
        for d in loads[c]:
            d.wait()
        bk = base + c * _CH
        ws = []
        for i in range(_NB):
            ws.append(pltpu.async_copy(
                a_v.at[s], out_hbm.at[pl.ds(bk, _CH), i, :, pl.ds(0, _F)],
                wsem))
            ws.append(pltpu.async_copy(
                b_v.at[s], out_hbm.at[pl.ds(bk, _CH), :, i, pl.ds(_F, _F)],
                wsem))
        writes[c] = ws
    for c in range(max(0, _NCHUNK - _SLOTS + 1), _NCHUNK):
        for d in writes[c]:
            d.wait()


_sc_assemble = functools.partial(
    pl.kernel,
    out_type=jax.ShapeDtypeStruct((_ROWS, _NB, _NB, 2 * _F), jnp.float32),
    mesh=plsc.VectorSubcoreMesh(core_axis_name="c", subcore_axis_name="s"),
    scratch_types=[
        pltpu.VMEM((_SLOTS, _CH, _NB, _F), jnp.float32),
        pltpu.VMEM((_SLOTS, _CH, _NB, _F), jnp.float32),
        pltpu.SemaphoreType.DMA,
        pltpu.SemaphoreType.DMA,
    ],
)(_sc_body)


def kernel(x1, x2):
    nbatch, nk, nb, f = x1.shape
    rows = nbatch * nk
    out = _sc_assemble(x1.reshape(rows, nb, f), x2.reshape(rows, nb, f))
    return out.reshape(nbatch, nk, nb, nb, 2 * f)


# speedup vs baseline: 1.0314x; 1.0314x over previous
import functools

import jax
import jax.numpy as jnp
from jax import lax
from jax.experimental import pallas as pl
from jax.experimental.pallas import tpu as pltpu
from jax.experimental.pallas import tpu_sc as plsc

_NC = 2
_NS = 16
_NW = _NC * _NS

_ROWS = 4096
_NB = 8
_F = 128
_CH = 16
_SLOTS = 3
_PER_W = _ROWS // _NW
_NCHUNK = _PER_W // _CH


def _sc_body(x1_hbm, x2_hbm, out_hbm, a_v, b_v, lsem, wsem):
    wid = lax.axis_index("s") * _NC + lax.axis_index("c")
    base = wid * _PER_W

    def start_loads(c):
        s = c % _SLOTS
        bk = base + c * _CH
        return [
            pltpu.async_copy(x1_hbm.at[pl.ds(bk, _CH)], a_v.at[s], lsem),
            pltpu.async_copy(x2_hbm.at[pl.ds(bk, _CH)], b_v.at[s], lsem),
        ]

    loads = {0: start_loads(0)}
    writes = {}
    for c in range(_NCHUNK):
        s = c % _SLOTS
        if c + 1 - _SLOTS >= 0:
            for d in writes[c + 1 - _SLOTS]:
                d.wait()
        if c + 1 < _NCHUNK:
            loads[c + 1] = start_loads(c + 1)
        for d in loads[c]:
            d.wait()
        bk = base + c * _CH
        ws = []
        for i in range(_NB):
            ws.append(pltpu.async_copy(
                a_v.at[s], out_hbm.at[pl.ds(bk, _CH), i, :, pl.ds(0, _F)],
                wsem))
            ws.append(pltpu.async_copy(
                b_v.at[s], out_hbm.at[pl.ds(bk, _CH), :, i, pl.ds(_F, _F)],
                wsem))
        writes[c] = ws
    for c in range(max(0, _NCHUNK - _SLOTS + 1), _NCHUNK):
        for d in writes[c]:
            d.wait()


_sc_assemble = functools.partial(
    pl.kernel,
    out_type=jax.ShapeDtypeStruct((_ROWS, _NB, _NB, 2 * _F), jnp.float32),
    mesh=plsc.VectorSubcoreMesh(core_axis_name="c", subcore_axis_name="s"),
    scratch_types=[
        pltpu.VMEM((_SLOTS, _CH, _NB, _F), jnp.float32),
        pltpu.VMEM((_SLOTS, _CH, _NB, _F), jnp.float32),
        pltpu.SemaphoreType.DMA,
        pltpu.SemaphoreType.DMA,
    ],
)(_sc_body)


def kernel(x1, x2):
    nbatch, nk, nb, f = x1.shape
    rows = nbatch * nk
    out = _sc_assemble(x1.reshape(rows, nb, f), x2.reshape(rows, nb, f))
    return out.reshape(nbatch, nk, nb, nb, 2 * f)


# speedup vs baseline: 1.0617x; 1.0293x over previous
import functools

import jax
import jax.numpy as jnp
from jax import lax
from jax.experimental import pallas as pl
from jax.experimental.pallas import tpu as pltpu
from jax.experimental.pallas import tpu_sc as plsc

_NC = 2
_NS = 16
_NW = _NC * _NS

_ROWS = 4096
_NB = 8
_F = 128
_CH = 32
_SLOTS = 3
_PER_W = _ROWS // _NW
_NCHUNK = _PER_W // _CH


def _sc_body(x1_hbm, x2_hbm, out_hbm, v, lsem, wsem):
    wid = lax.axis_index("s") * _NC + lax.axis_index("c")
    base = wid * _PER_W

    jobs = []
    for src_is_x1 in (True, False):
        for c in range(_NCHUNK):
            jobs.append((src_is_x1, c))

    def start_load(t):
        src_is_x1, c = jobs[t]
        src = x1_hbm if src_is_x1 else x2_hbm
        return pltpu.async_copy(src.at[pl.ds(base + c * _CH, _CH)],
                                v.at[t % _SLOTS], lsem)

    loads = {0: start_load(0)}
    writes = {}
    for t in range(len(jobs)):
        s = t % _SLOTS
        if t + 1 - _SLOTS >= 0:
            for d in writes[t + 1 - _SLOTS]:
                d.wait()
        if t + 1 < len(jobs):
            loads[t + 1] = start_load(t + 1)
        loads[t].wait()
        src_is_x1, c = jobs[t]
        bk = base + c * _CH
        ws = []
        for r in range(_NB):
            if src_is_x1:
                dst = out_hbm.at[pl.ds(bk, _CH), r, :, pl.ds(0, _F)]
            else:
                dst = out_hbm.at[pl.ds(bk, _CH), :, r, pl.ds(_F, _F)]
            ws.append(pltpu.async_copy(v.at[s], dst, wsem))
        writes[t] = ws
    for t in range(max(0, len(jobs) - _SLOTS + 1), len(jobs)):
        for d in writes[t]:
            d.wait()


_sc_assemble = functools.partial(
    pl.kernel,
    out_type=jax.ShapeDtypeStruct((_ROWS, _NB, _NB, 2 * _F), jnp.float32),
    mesh=plsc.VectorSubcoreMesh(core_axis_name="c", subcore_axis_name="s"),
    scratch_types=[
        pltpu.VMEM((_SLOTS, _CH, _NB, _F), jnp.float32),
        pltpu.SemaphoreType.DMA,
        pltpu.SemaphoreType.DMA,
    ],
)(_sc_body)


def kernel(x1, x2):
    nbatch, nk, nb, f = x1.shape
    rows = nbatch * nk
    out = _sc_assemble(x1.reshape(rows, nb, f), x2.reshape(rows, nb, f))
    return out.reshape(nbatch, nk, nb, nb, 2 * f)
